# baseline (speedup 1.0000x reference)
import jax
import jax.numpy as jnp
from jax.experimental import pallas as pl
from jax.experimental.pallas import tpu as pltpu


def _bigru_kernel(xs_ref, ms_ref, wf_ref, ufT_ref, bf_ref,
                  wb_ref, ubT_ref, bb_ref, wof_ref, wob_ref, bout_ref,
                  out_ref, h_ref, pf_ref):
    S = xs_ref.shape[0]
    H = ufT_ref.shape[1]

    wf = wf_ref[:, :]
    ufT = ufT_ref[:, :]
    bf = bf_ref[:, :]
    wof = wof_ref[:, :]

    h_ref[:, :] = jnp.zeros_like(h_ref)

    def fwd(t, carry):
        x_t = xs_ref[pl.ds(t, 1), :]
        h = h_ref[:, :]
        g = wf * x_t + bf
        gh = jnp.dot(ufT, h.astype(jnp.bfloat16),
                     preferred_element_type=jnp.float32)
        z = jax.nn.sigmoid(g[0:H, :] + gh[0:H, :])
        r = jax.nn.sigmoid(g[H:2 * H, :] + gh[H:2 * H, :])
        c = jnp.tanh(g[2 * H:3 * H, :] + r * gh[2 * H:3 * H, :])
        hn = (1.0 - z) * h + z * c
        h_ref[:, :] = hn
        pf_ref[pl.ds(t, 1), :] = jnp.sum(hn * wof, axis=0, keepdims=True)
        return carry

    jax.lax.fori_loop(0, S, fwd, 0)

    wb = wb_ref[:, :]
    ubT = ubT_ref[:, :]
    bb = bb_ref[:, :]
    wob = wob_ref[:, :]
    bout = bout_ref[0, 0]

    h_ref[:, :] = jnp.zeros_like(h_ref)

    def bwd(i, carry):
        t = S - 1 - i
        x_t = xs_ref[pl.ds(t, 1), :]
        h = h_ref[:, :]
        g = wb * x_t + bb
        gh = jnp.dot(ubT, h.astype(jnp.bfloat16),
                     preferred_element_type=jnp.float32)
        z = jax.nn.sigmoid(g[0:H, :] + gh[0:H, :])
        r = jax.nn.sigmoid(g[H:2 * H, :] + gh[H:2 * H, :])
        c = jnp.tanh(g[2 * H:3 * H, :] + r * gh[2 * H:3 * H, :])
        hn = (1.0 - z) * h + z * c
        h_ref[:, :] = hn
        pb = jnp.sum(hn * wob, axis=0, keepdims=True)
        imp = pf_ref[pl.ds(t, 1), :] + pb + bout
        m = ms_ref[pl.ds(t, 1), :]
        out_ref[pl.ds(t, 1), :] = m * x_t + (1.0 - m) * imp
        return carry

    jax.lax.fori_loop(0, S, bwd, 0)


def kernel(x, mask, Wf, Uf, bf, Wb, Ub, bb, Wout, bout):
    B, S, N, C = x.shape
    H = Uf.shape[0]
    R = B * N
    RT = 2048
    G = R // RT

    xs = x.transpose(1, 0, 2, 3).reshape(S, R)
    ms = mask.astype(jnp.float32).transpose(1, 0, 2, 3).reshape(S, R)

    wf = Wf.reshape(3 * H, 1)
    wb = Wb.reshape(3 * H, 1)
    ufT = Uf.T.astype(jnp.bfloat16)
    ubT = Ub.T.astype(jnp.bfloat16)
    bf2 = bf.reshape(3 * H, 1)
    bb2 = bb.reshape(3 * H, 1)
    wof = Wout[:H, 0:1]
    wob = Wout[H:, 0:1]
    bout2 = bout.reshape(1, 1)

    full = lambda shape: pl.BlockSpec(shape, lambda i: (0, 0))
    tile = pl.BlockSpec((S, RT), lambda i: (0, i))

    out = pl.pallas_call(
        _bigru_kernel,
        grid=(G,),
        in_specs=[
            tile,
            tile,
            full((3 * H, 1)),
            full((3 * H, H)),
            full((3 * H, 1)),
            full((3 * H, 1)),
            full((3 * H, H)),
            full((3 * H, 1)),
            full((H, 1)),
            full((H, 1)),
            full((1, 1)),
        ],
        out_specs=tile,
        out_shape=jax.ShapeDtypeStruct((S, R), jnp.float32),
        scratch_shapes=[
            pltpu.VMEM((H, RT), jnp.float32),
            pltpu.VMEM((S, RT), jnp.float32),
        ],
        compiler_params=pltpu.CompilerParams(
            dimension_semantics=("arbitrary",),
        ),
    )(xs, ms, wf, ufT, bf2, wb, ubT, bb2, wof, wob, bout2)

    return out.reshape(S, B, N, C).transpose(1, 0, 2, 3)


# speedup vs baseline: 1.3419x; 1.3419x over previous
import jax
import jax.numpy as jnp
from jax.experimental import pallas as pl
from jax.experimental.pallas import tpu as pltpu

_K = 144


def _bigru_kernel(xs_ref, ms_ref, w_ref, wof_ref, wob_ref, bout_ref,
                  out_ref, hx_ref, hs_ref, pf_ref, pb_ref):
    S = xs_ref.shape[0]
    H = wof_ref.shape[0]
    RT = xs_ref.shape[1]

    wof = wof_ref[:, :]
    wob = wob_ref[:, :]
    w = w_ref[:, :]

    hs_ref[:, :] = jnp.zeros_like(hs_ref)
    hx_ref[:, :] = jnp.zeros_like(hx_ref)
    ones_pad = jnp.concatenate(
        [jnp.ones((1, RT), jnp.float32), jnp.zeros((1, RT), jnp.float32)],
        axis=0)
    hx_ref[pl.ds(2 * H + 2, 2), :] = ones_pad.astype(jnp.bfloat16)

    def step(t, carry):
        tb = S - 1 - t
        xf = xs_ref[pl.ds(t, 1), :]
        xb = xs_ref[pl.ds(tb, 1), :]
        hx_ref[pl.ds(2 * H, 2), :] = jnp.concatenate(
            [xf, xb], axis=0).astype(jnp.bfloat16)
        gates = jnp.dot(w, hx_ref[:, :],
                        preferred_element_type=jnp.float32)
        h = hs_ref[:, :]

        zr_f = 0.5 * jnp.tanh(0.5 * gates[0:2 * H, :]) + 0.5
        cf = jnp.tanh(gates[3 * H:4 * H, :]
                      + zr_f[H:2 * H, :] * gates[2 * H:3 * H, :])
        hf = h[0:H, :]
        hfn = hf + zr_f[0:H, :] * (cf - hf)

        zr_b = 0.5 * jnp.tanh(0.5 * gates[4 * H:6 * H, :]) + 0.5
        cb = jnp.tanh(gates[7 * H:8 * H, :]
                      + zr_b[H:2 * H, :] * gates[6 * H:7 * H, :])
        hb = h[H:2 * H, :]
        hbn = hb + zr_b[0:H, :] * (cb - hb)

        hn = jnp.concatenate([hfn, hbn], axis=0)
        hs_ref[:, :] = hn
        hx_ref[pl.ds(0, 2 * H), :] = hn.astype(jnp.bfloat16)

        pf_ref[pl.ds(t, 1), :] = jnp.sum(hfn * wof, axis=0, keepdims=True)
        pb_ref[pl.ds(tb, 1), :] = jnp.sum(hbn * wob, axis=0, keepdims=True)
        return carry

    jax.lax.fori_loop(0, S, step, 0)

    xs = xs_ref[:, :]
    m = ms_ref[:, :]
    imp = pf_ref[:, :] + pb_ref[:, :] + bout_ref[0, 0]
    out_ref[:, :] = m * xs + (1.0 - m) * imp


def _pack_weights(Wf, Uf, bf, Wb, Ub, bb, H):
    w = jnp.zeros((8 * H, _K), jnp.float32)
    UfT, UbT = Uf.T, Ub.T
    w = w.at[0:2 * H, 0:H].set(UfT[0:2 * H, :])
    w = w.at[2 * H:3 * H, 0:H].set(UfT[2 * H:3 * H, :])
    w = w.at[4 * H:6 * H, H:2 * H].set(UbT[0:2 * H, :])
    w = w.at[6 * H:7 * H, H:2 * H].set(UbT[2 * H:3 * H, :])
    w = w.at[0:2 * H, 2 * H].set(Wf[0, 0:2 * H])
    w = w.at[3 * H:4 * H, 2 * H].set(Wf[0, 2 * H:3 * H])
    w = w.at[4 * H:6 * H, 2 * H + 1].set(Wb[0, 0:2 * H])
    w = w.at[7 * H:8 * H, 2 * H + 1].set(Wb[0, 2 * H:3 * H])
    w = w.at[0:2 * H, 2 * H + 2].set(bf[0:2 * H])
    w = w.at[3 * H:4 * H, 2 * H + 2].set(bf[2 * H:3 * H])
    w = w.at[4 * H:6 * H, 2 * H + 2].set(bb[0:2 * H])
    w = w.at[7 * H:8 * H, 2 * H + 2].set(bb[2 * H:3 * H])
    return w.astype(jnp.bfloat16)


def kernel(x, mask, Wf, Uf, bf, Wb, Ub, bb, Wout, bout):
    B, S, N, C = x.shape
    H = Uf.shape[0]
    R = B * N
    RT = 2048
    G = R // RT

    xs = x.transpose(1, 0, 2, 3).reshape(S, R)
    ms = mask.astype(jnp.float32).transpose(1, 0, 2, 3).reshape(S, R)

    w = _pack_weights(Wf, Uf, bf, Wb, Ub, bb, H)
    wof = Wout[:H, 0:1]
    wob = Wout[H:, 0:1]
    bout2 = bout.reshape(1, 1)

    full = lambda shape: pl.BlockSpec(shape, lambda i: (0, 0))
    tile = pl.BlockSpec((S, RT), lambda i: (0, i))

    out = pl.pallas_call(
        _bigru_kernel,
        grid=(G,),
        in_specs=[
            tile,
            tile,
            full((8 * H, _K)),
            full((H, 1)),
            full((H, 1)),
            full((1, 1)),
        ],
        out_specs=tile,
        out_shape=jax.ShapeDtypeStruct((S, R), jnp.float32),
        scratch_shapes=[
            pltpu.VMEM((_K, RT), jnp.bfloat16),
            pltpu.VMEM((2 * H, RT), jnp.float32),
            pltpu.VMEM((S, RT), jnp.float32),
            pltpu.VMEM((S, RT), jnp.float32),
        ],
        compiler_params=pltpu.CompilerParams(
            dimension_semantics=("arbitrary",),
        ),
    )(xs, ms, w, wof, wob, bout2)

    return out.reshape(S, B, N, C).transpose(1, 0, 2, 3)
